# Initial kernel scaffold; baseline (speedup 1.0000x reference)
import functools

import jax
import jax.numpy as jnp
from jax import lax
from jax.experimental import pallas as pl
from jax.experimental.pallas import tpu as pltpu
from jax.experimental.pallas import tpu_sc as plsc

H = 128
NUM_POS = 25000
NUM_LIT = 50000
NUM_CLAUSE = 25000
E = 300000
NUM_ROUND = 4

NC = 2
NS = 16
CHUNK = 128
SEG_OUT = 25000
HALF = SEG_OUT // NC
ACC_ROWS = 12544
FULL_OUT_CHUNKS = HALF // CHUNK
OUT_REM = HALF - FULL_OUT_CHUNKS * CHUNK

_MESH = plsc.VectorSubcoreMesh(
    core_axis_name="c", subcore_axis_name="s", num_cores=NC, num_subcores=NS)


def _pad_edges(n_edges):
    per = NS * CHUNK
    return ((n_edges + per - 1) // per) * per


def _make_seg_sum(n_edges_padded):
    chunks_per_tec = n_edges_padded // (NS * CHUNK)

    @functools.partial(
        pl.kernel,
        out_type=jax.ShapeDtypeStruct((SEG_OUT, H), jnp.float32),
        mesh=_MESH,
        scratch_types=[
            pltpu.VMEM((CHUNK,), jnp.int32),
            pltpu.VMEM((CHUNK,), jnp.int32),
            pltpu.VMEM((CHUNK,), jnp.int32),
            pltpu.VMEM((CHUNK, H), jnp.float32),
            pltpu.VMEM_SHARED((ACC_ROWS, H), jnp.float32),
            pltpu.SemaphoreType.DMA,
        ],
    )
    def seg_sum(table_hbm, src_hbm, dst_hbm, out_hbm,
                src_v, dst_v, loc_v, rows_v, acc_sh, sem):
        cid = lax.axis_index("c")
        sid = lax.axis_index("s")
        lo = cid * HALF

        @pl.loop(0, CHUNK)
        def _(r):
            @pl.loop(0, H // 16)
            def _(j):
                rows_v[r, pl.ds(j * 16, 16)] = jnp.zeros((16,), jnp.float32)

        @pl.loop(sid, ACC_ROWS // CHUNK, step=NS)
        def _(cidx):
            pltpu.sync_copy(rows_v, acc_sh.at[pl.ds(cidx * CHUNK, CHUNK)])

        plsc.subcore_barrier()

        base = sid * (chunks_per_tec * CHUNK)

        @pl.loop(0, chunks_per_tec)
        def _(t):
            e0 = base + t * CHUNK
            pltpu.sync_copy(src_hbm.at[pl.ds(e0, CHUNK)], src_v)
            pltpu.sync_copy(dst_hbm.at[pl.ds(e0, CHUNK)], dst_v)
            for j in range(CHUNK // 16):
                d = dst_v[pl.ds(j * 16, 16)] - lo
                oob = (d < 0) | (d >= HALF)
                loc_v[pl.ds(j * 16, 16)] = jnp.where(oob, HALF, d)
            pltpu.async_copy(table_hbm.at[src_v], rows_v, sem).wait()
            pltpu.sync_copy(rows_v, acc_sh.at[loc_v], add=True)

        plsc.subcore_barrier()

        @pl.loop(sid, FULL_OUT_CHUNKS, step=NS)
        def _(cidx):
            pltpu.sync_copy(acc_sh.at[pl.ds(cidx * CHUNK, CHUNK)],
                            out_hbm.at[pl.ds(lo + cidx * CHUNK, CHUNK)])

        @pl.when(sid == 0)
        def _():
            pltpu.sync_copy(
                acc_sh.at[pl.ds(FULL_OUT_CHUNKS * CHUNK, OUT_REM)],
                out_hbm.at[pl.ds(lo + FULL_OUT_CHUNKS * CHUNK, OUT_REM)])

    return seg_sum


E1P = _pad_edges(E)
E2P = _pad_edges(2 * E)
_seg_sum_1 = _make_seg_sum(E1P)
_seg_sum_2 = _make_seg_sum(E2P)


BLK = 1000


def _mlp_body(x_ref, w_ref, b_ref, o_ref):
    h = x_ref[...]
    b = b_ref[...]
    for i in range(3):
        h = lax.dot_general(h, w_ref[i], (((1,), (1,)), ((), ())),
                            preferred_element_type=jnp.float32)
        h = h + b[i][None, :]
        if i < 2:
            h = jnp.maximum(h, 0.0)
    o_ref[...] = h


def _mlp(x, W, b):
    n = x.shape[0]
    return pl.pallas_call(
        _mlp_body,
        grid=(n // BLK,),
        in_specs=[
            pl.BlockSpec((BLK, H), lambda i: (i, 0)),
            pl.BlockSpec((3, H, H), lambda i: (0, 0, 0)),
            pl.BlockSpec((3, H), lambda i: (0, 0)),
        ],
        out_specs=pl.BlockSpec((BLK, H), lambda i: (i, 0)),
        out_shape=jax.ShapeDtypeStruct((n, H), jnp.float32),
        compiler_params=pltpu.CompilerParams(
            dimension_semantics=("parallel",)),
    )(x, W, b)


def _gru_gates(gi, gh, h):
    r = jax.nn.sigmoid(gi[:, :H] + gh[:, :H])
    z = jax.nn.sigmoid(gi[:, H:2 * H] + gh[:, H:2 * H])
    n = jnp.tanh(gi[:, 2 * H:] + r * gh[:, 2 * H:])
    return (1.0 - z) * n + z * h


def _gru_c_body(x_ref, h_ref, wih_ref, whh_ref, bih_ref, bhh_ref, o_ref):
    x = x_ref[...]
    h = h_ref[...]
    gi = lax.dot_general(x, wih_ref[...], (((1,), (1,)), ((), ())),
                         preferred_element_type=jnp.float32) + bih_ref[...]
    gh = lax.dot_general(h, whh_ref[...], (((1,), (1,)), ((), ())),
                         preferred_element_type=jnp.float32) + bhh_ref[...]
    o_ref[...] = _gru_gates(gi, gh, h)


def _gru_c(x, h, Wih, Whh, bih, bhh):
    n = x.shape[0]
    return pl.pallas_call(
        _gru_c_body,
        grid=(n // BLK,),
        in_specs=[
            pl.BlockSpec((BLK, H), lambda i: (i, 0)),
            pl.BlockSpec((BLK, H), lambda i: (i, 0)),
            pl.BlockSpec((3 * H, H), lambda i: (0, 0)),
            pl.BlockSpec((3 * H, H), lambda i: (0, 0)),
            pl.BlockSpec((1, 3 * H), lambda i: (0, 0)),
            pl.BlockSpec((1, 3 * H), lambda i: (0, 0)),
        ],
        out_specs=pl.BlockSpec((BLK, H), lambda i: (i, 0)),
        out_shape=jax.ShapeDtypeStruct((n, H), jnp.float32),
        compiler_params=pltpu.CompilerParams(
            dimension_semantics=("parallel",)),
    )(x, h, Wih, Whh, bih, bhh)


def _gru_l_body(x1_ref, x2_ref, h_ref, wih_ref, whh_ref, bih_ref, bhh_ref,
                o_ref):
    h = h_ref[...]
    w = wih_ref[...]
    gi = (lax.dot_general(x1_ref[...], w[:, :H], (((1,), (1,)), ((), ())),
                          preferred_element_type=jnp.float32)
          + lax.dot_general(x2_ref[...], w[:, H:], (((1,), (1,)), ((), ())),
                            preferred_element_type=jnp.float32)
          + bih_ref[...])
    gh = lax.dot_general(h, whh_ref[...], (((1,), (1,)), ((), ())),
                         preferred_element_type=jnp.float32) + bhh_ref[...]
    o_ref[...] = _gru_gates(gi, gh, h)


def _gru_l(x1, x2, h, Wih, Whh, bih, bhh):
    n = x1.shape[0]
    return pl.pallas_call(
        _gru_l_body,
        grid=(n // BLK,),
        in_specs=[
            pl.BlockSpec((BLK, H), lambda i: (i, 0)),
            pl.BlockSpec((BLK, H), lambda i: (i, 0)),
            pl.BlockSpec((BLK, H), lambda i: (i, 0)),
            pl.BlockSpec((3 * H, 2 * H), lambda i: (0, 0)),
            pl.BlockSpec((3 * H, H), lambda i: (0, 0)),
            pl.BlockSpec((1, 3 * H), lambda i: (0, 0)),
            pl.BlockSpec((1, 3 * H), lambda i: (0, 0)),
        ],
        out_specs=pl.BlockSpec((BLK, H), lambda i: (i, 0)),
        out_shape=jax.ShapeDtypeStruct((n, H), jnp.float32),
        compiler_params=pltpu.CompilerParams(
            dimension_semantics=("parallel",)),
    )(x1, x2, h, Wih, Whh, bih, bhh)


def kernel(l_embedding, c_embedding, pos_edge_index, neg_edge_index,
           l_mlp_W, l_mlp_b, c_mlp_W, c_mlp_b,
           l_gru_Wih, l_gru_Whh, l_gru_bih, l_gru_bhh,
           c_gru_Wih, c_gru_Whh, c_gru_bih, c_gru_bhh):
    ps = pos_edge_index[0].astype(jnp.int32)
    pd = pos_edge_index[1].astype(jnp.int32)
    ns = neg_edge_index[0].astype(jnp.int32)
    nd = neg_edge_index[1].astype(jnp.int32)

    def pad_pair(src, dst, n_pad):
        extra = n_pad - src.shape[0]
        src_p = jnp.concatenate([src, jnp.zeros((extra,), jnp.int32)])
        dst_p = jnp.concatenate([dst, jnp.full((extra,), SEG_OUT, jnp.int32)])
        return src_p, dst_p

    l2c_src, l2c_dst = pad_pair(
        jnp.concatenate([ps, ns + NUM_POS]), jnp.concatenate([pd, nd]), E2P)
    c2lp_src, c2lp_dst = pad_pair(pd, ps, E1P)
    c2ln_src, c2ln_dst = pad_pair(nd, ns, E1P)

    bih_l = l_gru_bih.reshape(1, 3 * H)
    bhh_l = l_gru_bhh.reshape(1, 3 * H)
    bih_c = c_gru_bih.reshape(1, 3 * H)
    bhh_c = c_gru_bhh.reshape(1, 3 * H)

    l_emb, c_emb = l_embedding, c_embedding
    for _ in range(NUM_ROUND):
        l_msg = _mlp(l_emb, l_mlp_W, l_mlp_b)
        c_msg = _mlp(c_emb, c_mlp_W, c_mlp_b)
        l2c = _seg_sum_2(l_msg, l2c_src, l2c_dst)
        c2l_p = _seg_sum_1(c_msg, c2lp_src, c2lp_dst)
        c2l_n = _seg_sum_1(c_msg, c2ln_src, c2ln_dst)
        c2l = jnp.concatenate([c2l_p, c2l_n], axis=0)
        flip = jnp.concatenate([l_emb[NUM_POS:], l_emb[:NUM_POS]], axis=0)
        c_emb = _gru_c(l2c, c_emb, c_gru_Wih, c_gru_Whh, bih_c, bhh_c)
        l_emb = _gru_l(c2l, flip, l_emb, l_gru_Wih, l_gru_Whh, bih_l, bhh_l)
    return (l_emb, c_emb)


# speedup vs baseline: 3.2668x; 3.2668x over previous
import functools

import jax
import jax.numpy as jnp
from jax import lax
from jax.experimental import pallas as pl
from jax.experimental.pallas import tpu as pltpu
from jax.experimental.pallas import tpu_sc as plsc

H = 128
NUM_POS = 25000
NUM_LIT = 50000
NUM_CLAUSE = 25000
E = 300000
NUM_ROUND = 4

NC = 2
NS = 16
CHUNK = 128
SEG_OUT = 25000
HALF0 = 12544
HALF1 = SEG_OUT - HALF0
TRASH = HALF0
ACC_ROWS = 12672
REM1 = HALF1 - (HALF1 // CHUNK) * CHUNK

_MESH = plsc.VectorSubcoreMesh(
    core_axis_name="c", subcore_axis_name="s", num_cores=NC, num_subcores=NS)


def _pad_edges(n_edges):
    per = NS * CHUNK
    return ((n_edges + per - 1) // per) * per


def _make_seg_sum(n_edges_padded):
    chunks_per_tec = n_edges_padded // (NS * CHUNK)

    @functools.partial(
        pl.kernel,
        out_type=jax.ShapeDtypeStruct((SEG_OUT, H), jnp.float32),
        mesh=_MESH,
        scratch_types=[
            pltpu.VMEM((CHUNK,), jnp.int32),
            pltpu.VMEM((CHUNK,), jnp.int32),
            pltpu.VMEM((CHUNK,), jnp.int32),
            pltpu.VMEM((CHUNK, H), jnp.float32),
            pltpu.VMEM_SHARED((ACC_ROWS, H), jnp.float32),
            pltpu.SemaphoreType.DMA,
        ],
    )
    def seg_sum(table_hbm, src_hbm, dst_hbm, out_hbm,
                src_v, dst_v, loc_v, rows_v, acc_sh, sem):
        cid = lax.axis_index("c")
        sid = lax.axis_index("s")
        lo = cid * HALF0
        cnt = jnp.where(cid == 0, HALF0, HALF1)

        @pl.loop(0, CHUNK)
        def _(r):
            @pl.loop(0, H // 16)
            def _(j):
                rows_v[r, pl.ds(j * 16, 16)] = jnp.zeros((16,), jnp.float32)

        @pl.loop(sid, ACC_ROWS // CHUNK, step=NS)
        def _(cidx):
            pltpu.sync_copy(rows_v, acc_sh.at[pl.ds(cidx * CHUNK, CHUNK)])

        plsc.subcore_barrier()

        base = sid * (chunks_per_tec * CHUNK)

        @pl.loop(0, chunks_per_tec)
        def _(t):
            e0 = base + t * CHUNK
            pltpu.sync_copy(src_hbm.at[pl.ds(e0, CHUNK)], src_v)
            pltpu.sync_copy(dst_hbm.at[pl.ds(e0, CHUNK)], dst_v)
            for j in range(CHUNK // 16):
                d = dst_v[pl.ds(j * 16, 16)] - lo
                oob = (d < 0) | (d >= cnt)
                loc_v[pl.ds(j * 16, 16)] = jnp.where(oob, TRASH, d)
            pltpu.async_copy(table_hbm.at[src_v], rows_v, sem).wait()
            pltpu.sync_copy(rows_v, acc_sh.at[loc_v], add=True)

        plsc.subcore_barrier()

        nfull = cnt // CHUNK

        @pl.loop(sid, nfull, step=NS)
        def _(cidx):
            pltpu.sync_copy(acc_sh.at[pl.ds(cidx * CHUNK, CHUNK)],
                            out_hbm.at[pl.ds(lo + cidx * CHUNK, CHUNK)])

        @pl.when((cid == 1) & (sid == 0))
        def _():
            pltpu.sync_copy(
                acc_sh.at[pl.ds((HALF1 // CHUNK) * CHUNK, REM1)],
                out_hbm.at[pl.ds(HALF0 + (HALF1 // CHUNK) * CHUNK, REM1)])

    return seg_sum


E1P = _pad_edges(E)
E2P = _pad_edges(2 * E)
_seg_sum_1 = _make_seg_sum(E1P)
_seg_sum_2 = _make_seg_sum(E2P)


BLK = 1000


def _mlp_body(x_ref, w_ref, b_ref, o_ref):
    h = x_ref[...]
    b = b_ref[...]
    for i in range(3):
        h = lax.dot_general(h, w_ref[i], (((1,), (1,)), ((), ())),
                            preferred_element_type=jnp.float32)
        h = h + b[i][None, :]
        if i < 2:
            h = jnp.maximum(h, 0.0)
    o_ref[...] = h


def _mlp(x, W, b):
    n = x.shape[0]
    return pl.pallas_call(
        _mlp_body,
        grid=(n // BLK,),
        in_specs=[
            pl.BlockSpec((BLK, H), lambda i: (i, 0)),
            pl.BlockSpec((3, H, H), lambda i: (0, 0, 0)),
            pl.BlockSpec((3, H), lambda i: (0, 0)),
        ],
        out_specs=pl.BlockSpec((BLK, H), lambda i: (i, 0)),
        out_shape=jax.ShapeDtypeStruct((n, H), jnp.float32),
        compiler_params=pltpu.CompilerParams(
            dimension_semantics=("parallel",)),
    )(x, W, b)


def _gru_gates(gi, gh, h):
    r = jax.nn.sigmoid(gi[:, :H] + gh[:, :H])
    z = jax.nn.sigmoid(gi[:, H:2 * H] + gh[:, H:2 * H])
    n = jnp.tanh(gi[:, 2 * H:] + r * gh[:, 2 * H:])
    return (1.0 - z) * n + z * h


def _gru_c_body(x_ref, h_ref, wih_ref, whh_ref, bih_ref, bhh_ref, o_ref):
    x = x_ref[...]
    h = h_ref[...]
    gi = lax.dot_general(x, wih_ref[...], (((1,), (1,)), ((), ())),
                         preferred_element_type=jnp.float32) + bih_ref[...]
    gh = lax.dot_general(h, whh_ref[...], (((1,), (1,)), ((), ())),
                         preferred_element_type=jnp.float32) + bhh_ref[...]
    o_ref[...] = _gru_gates(gi, gh, h)


def _gru_c(x, h, Wih, Whh, bih, bhh):
    n = x.shape[0]
    return pl.pallas_call(
        _gru_c_body,
        grid=(n // BLK,),
        in_specs=[
            pl.BlockSpec((BLK, H), lambda i: (i, 0)),
            pl.BlockSpec((BLK, H), lambda i: (i, 0)),
            pl.BlockSpec((3 * H, H), lambda i: (0, 0)),
            pl.BlockSpec((3 * H, H), lambda i: (0, 0)),
            pl.BlockSpec((1, 3 * H), lambda i: (0, 0)),
            pl.BlockSpec((1, 3 * H), lambda i: (0, 0)),
        ],
        out_specs=pl.BlockSpec((BLK, H), lambda i: (i, 0)),
        out_shape=jax.ShapeDtypeStruct((n, H), jnp.float32),
        compiler_params=pltpu.CompilerParams(
            dimension_semantics=("parallel",)),
    )(x, h, Wih, Whh, bih, bhh)


def _gru_l_body(x1_ref, x2_ref, h_ref, wih_ref, whh_ref, bih_ref, bhh_ref,
                o_ref):
    h = h_ref[...]
    w = wih_ref[...]
    gi = (lax.dot_general(x1_ref[...], w[:, :H], (((1,), (1,)), ((), ())),
                          preferred_element_type=jnp.float32)
          + lax.dot_general(x2_ref[...], w[:, H:], (((1,), (1,)), ((), ())),
                            preferred_element_type=jnp.float32)
          + bih_ref[...])
    gh = lax.dot_general(h, whh_ref[...], (((1,), (1,)), ((), ())),
                         preferred_element_type=jnp.float32) + bhh_ref[...]
    o_ref[...] = _gru_gates(gi, gh, h)


def _gru_l(x1, x2, h, Wih, Whh, bih, bhh):
    n = x1.shape[0]
    return pl.pallas_call(
        _gru_l_body,
        grid=(n // BLK,),
        in_specs=[
            pl.BlockSpec((BLK, H), lambda i: (i, 0)),
            pl.BlockSpec((BLK, H), lambda i: (i, 0)),
            pl.BlockSpec((BLK, H), lambda i: (i, 0)),
            pl.BlockSpec((3 * H, 2 * H), lambda i: (0, 0)),
            pl.BlockSpec((3 * H, H), lambda i: (0, 0)),
            pl.BlockSpec((1, 3 * H), lambda i: (0, 0)),
            pl.BlockSpec((1, 3 * H), lambda i: (0, 0)),
        ],
        out_specs=pl.BlockSpec((BLK, H), lambda i: (i, 0)),
        out_shape=jax.ShapeDtypeStruct((n, H), jnp.float32),
        compiler_params=pltpu.CompilerParams(
            dimension_semantics=("parallel",)),
    )(x1, x2, h, Wih, Whh, bih, bhh)


def kernel(l_embedding, c_embedding, pos_edge_index, neg_edge_index,
           l_mlp_W, l_mlp_b, c_mlp_W, c_mlp_b,
           l_gru_Wih, l_gru_Whh, l_gru_bih, l_gru_bhh,
           c_gru_Wih, c_gru_Whh, c_gru_bih, c_gru_bhh):
    ps = pos_edge_index[0].astype(jnp.int32)
    pd = pos_edge_index[1].astype(jnp.int32)
    ns = neg_edge_index[0].astype(jnp.int32)
    nd = neg_edge_index[1].astype(jnp.int32)

    def pad_pair(src, dst, n_pad):
        extra = n_pad - src.shape[0]
        src_p = jnp.concatenate([src, jnp.zeros((extra,), jnp.int32)])
        dst_p = jnp.concatenate([dst, jnp.full((extra,), SEG_OUT, jnp.int32)])
        return src_p, dst_p

    l2c_src, l2c_dst = pad_pair(
        jnp.concatenate([ps, ns + NUM_POS]), jnp.concatenate([pd, nd]), E2P)
    c2lp_src, c2lp_dst = pad_pair(pd, ps, E1P)
    c2ln_src, c2ln_dst = pad_pair(nd, ns, E1P)

    bih_l = l_gru_bih.reshape(1, 3 * H)
    bhh_l = l_gru_bhh.reshape(1, 3 * H)
    bih_c = c_gru_bih.reshape(1, 3 * H)
    bhh_c = c_gru_bhh.reshape(1, 3 * H)

    l_emb, c_emb = l_embedding, c_embedding
    for _ in range(NUM_ROUND):
        l_msg = _mlp(l_emb, l_mlp_W, l_mlp_b)
        c_msg = _mlp(c_emb, c_mlp_W, c_mlp_b)
        l2c = _seg_sum_2(l_msg, l2c_src, l2c_dst)
        c2l_p = _seg_sum_1(c_msg, c2lp_src, c2lp_dst)
        c2l_n = _seg_sum_1(c_msg, c2ln_src, c2ln_dst)
        c2l = jnp.concatenate([c2l_p, c2l_n], axis=0)
        flip = jnp.concatenate([l_emb[NUM_POS:], l_emb[:NUM_POS]], axis=0)
        c_emb = _gru_c(l2c, c_emb, c_gru_Wih, c_gru_Whh, bih_c, bhh_c)
        l_emb = _gru_l(c2l, flip, l_emb, l_gru_Wih, l_gru_Whh, bih_l, bhh_l)
    return (l_emb, c_emb)


# speedup vs baseline: 4.9861x; 1.5263x over previous
import functools

import jax
import jax.numpy as jnp
from jax import lax
from jax.experimental import pallas as pl
from jax.experimental.pallas import tpu as pltpu
from jax.experimental.pallas import tpu_sc as plsc

H = 128
NUM_POS = 25000
NUM_LIT = 50000
NUM_CLAUSE = 25000
E = 300000
NUM_ROUND = 4

NC = 2
NS = 16
CHUNK = 96
WCHUNK = 128
SEG_OUT = 25000
HALF0 = 12544
HALF1 = SEG_OUT - HALF0
TRASH = HALF0
ACC_ROWS = 12672
REM1 = HALF1 - (HALF1 // WCHUNK) * WCHUNK

_MESH = plsc.VectorSubcoreMesh(
    core_axis_name="c", subcore_axis_name="s", num_cores=NC, num_subcores=NS)


def _pad_edges(n_edges):
    per = NS * 2 * CHUNK
    return ((n_edges + per - 1) // per) * per


def _make_seg_sum(n_edges_padded):
    edges_per_tec = n_edges_padded // NS
    NB = edges_per_tec // CHUNK

    @functools.partial(
        pl.kernel,
        out_type=jax.ShapeDtypeStruct((SEG_OUT, H), jnp.float32),
        mesh=_MESH,
        scratch_types=[
            pltpu.VMEM((2 * CHUNK,), jnp.int32),
            pltpu.VMEM((1, CHUNK), jnp.int32),
            pltpu.VMEM((1, CHUNK), jnp.int32),
            pltpu.VMEM((2 * CHUNK,), jnp.int32),
            pltpu.VMEM((1, CHUNK), jnp.int32),
            pltpu.VMEM((1, CHUNK), jnp.int32),
            pltpu.VMEM((CHUNK, H), jnp.float32),
            pltpu.VMEM((CHUNK, H), jnp.float32),
            pltpu.VMEM_SHARED((ACC_ROWS, H), jnp.float32),
            pltpu.SemaphoreType.DMA, pltpu.SemaphoreType.DMA,
            pltpu.SemaphoreType.DMA, pltpu.SemaphoreType.DMA,
            pltpu.SemaphoreType.DMA, pltpu.SemaphoreType.DMA,
        ],
    )
    def seg_sum(table_hbm, src_hbm, dst_hbm, out_hbm,
                stage0, src0, loc0, stage1, src1, loc1, rows0, rows1, acc_sh,
                semi0, semg0, sems0, semi1, semg1, sems1):
        cid = lax.axis_index("c")
        sid = lax.axis_index("s")
        lo = cid * HALF0
        cnt = jnp.where(cid == 0, HALF0, HALF1)
        nfull = cnt // WCHUNK
        sets = ((stage0, src0, loc0, rows0, semi0, semg0, sems0),
                (stage1, src1, loc1, rows1, semi1, semg1, sems1))
        base_edge = sid * edges_per_tec

        @pl.loop(0, CHUNK)
        def _(r):
            for j in range(H // 16):
                rows0[r, pl.ds(j * 16, 16)] = jnp.zeros((16,), jnp.float32)

        @pl.loop(sid, ACC_ROWS // CHUNK, step=NS)
        def _(cidx):
            pltpu.sync_copy(rows0, acc_sh.at[pl.ds(cidx * CHUNK, CHUNK)])

        plsc.subcore_barrier()

        def fire_idx(s, e0):
            stage, _, _, _, semi, _, _ = sets[s]
            pltpu.async_copy(src_hbm.at[pl.ds(e0, CHUNK)],
                             stage.at[pl.ds(0, CHUNK)], semi)
            pltpu.async_copy(dst_hbm.at[pl.ds(e0, CHUNK)],
                             stage.at[pl.ds(CHUNK, CHUNK)], semi)

        def wait_idx(s, e0):
            stage, _, _, _, semi, _, _ = sets[s]
            pltpu.make_async_copy(src_hbm.at[pl.ds(e0, CHUNK)],
                                  stage.at[pl.ds(0, CHUNK)], semi).wait()
            pltpu.make_async_copy(dst_hbm.at[pl.ds(e0, CHUNK)],
                                  stage.at[pl.ds(CHUNK, CHUNK)], semi).wait()

        def compute_and_gather(s):
            stage, srcg, locg, rows, _, semg, _ = sets[s]
            for i in range(CHUNK // 16):
                srcg[0, pl.ds(i * 16, 16)] = stage[pl.ds(i * 16, 16)]
                d = stage[pl.ds(CHUNK + i * 16, 16)] - lo
                oob = (d < 0) | (d >= cnt)
                locg[0, pl.ds(i * 16, 16)] = jnp.where(oob, TRASH, d)
            pltpu.async_copy(table_hbm.at[srcg.at[0]], rows, semg)

        def drain_scatter(s):
            _, _, locg, rows, _, _, sems = sets[s]
            pltpu.make_async_copy(rows, acc_sh.at[locg.at[0]], sems).wait()

        def wait_gather_fire_scatter(s):
            _, srcg, locg, rows, _, semg, sems = sets[s]
            pltpu.make_async_copy(table_hbm.at[srcg.at[0]], rows,
                                  semg).wait()
            pltpu.async_copy(rows, acc_sh.at[locg.at[0]], sems, add=True)

        fire_idx(0, base_edge)
        fire_idx(1, base_edge + CHUNK)

        @pl.loop(0, NB // 2)
        def _(t):
            e0 = base_edge + (2 * t) * CHUNK

            @pl.when(t > 0)
            def _():
                drain_scatter(0)
            wait_idx(0, e0)
            compute_and_gather(0)
            fire_idx(0, jnp.minimum(e0 + 2 * CHUNK,
                                    base_edge + edges_per_tec - CHUNK))

            @pl.when(t > 0)
            def _():
                wait_gather_fire_scatter(1)

            @pl.when(t > 0)
            def _():
                drain_scatter(1)
            wait_idx(1, e0 + CHUNK)
            compute_and_gather(1)
            fire_idx(1, jnp.minimum(e0 + 3 * CHUNK,
                                    base_edge + edges_per_tec - CHUNK))
            wait_gather_fire_scatter(0)

        wait_gather_fire_scatter(1)
        drain_scatter(0)
        drain_scatter(1)
        wait_idx(0, base_edge + edges_per_tec - CHUNK)
        wait_idx(1, base_edge + edges_per_tec - CHUNK)

        plsc.subcore_barrier()

        @pl.loop(sid, nfull, step=NS)
        def _(cidx):
            pltpu.sync_copy(acc_sh.at[pl.ds(cidx * WCHUNK, WCHUNK)],
                            out_hbm.at[pl.ds(lo + cidx * WCHUNK, WCHUNK)])

        @pl.when((cid == 1) & (sid == 0))
        def _():
            pltpu.sync_copy(
                acc_sh.at[pl.ds((HALF1 // WCHUNK) * WCHUNK, REM1)],
                out_hbm.at[pl.ds(HALF0 + (HALF1 // WCHUNK) * WCHUNK, REM1)])

    return seg_sum


E1P = _pad_edges(E)
E2P = _pad_edges(2 * E)
_seg_sum_1 = _make_seg_sum(E1P)
_seg_sum_2 = _make_seg_sum(E2P)


BLK = 1000


def _mlp_body(x_ref, w_ref, b_ref, o_ref):
    h = x_ref[...]
    b = b_ref[...]
    for i in range(3):
        h = lax.dot_general(h, w_ref[i], (((1,), (1,)), ((), ())),
                            preferred_element_type=jnp.float32)
        h = h + b[i][None, :]
        if i < 2:
            h = jnp.maximum(h, 0.0)
    o_ref[...] = h


def _mlp(x, W, b):
    n = x.shape[0]
    return pl.pallas_call(
        _mlp_body,
        grid=(n // BLK,),
        in_specs=[
            pl.BlockSpec((BLK, H), lambda i: (i, 0)),
            pl.BlockSpec((3, H, H), lambda i: (0, 0, 0)),
            pl.BlockSpec((3, H), lambda i: (0, 0)),
        ],
        out_specs=pl.BlockSpec((BLK, H), lambda i: (i, 0)),
        out_shape=jax.ShapeDtypeStruct((n, H), jnp.float32),
        compiler_params=pltpu.CompilerParams(
            dimension_semantics=("parallel",)),
    )(x, W, b)


def _gru_gates(gi, gh, h):
    r = jax.nn.sigmoid(gi[:, :H] + gh[:, :H])
    z = jax.nn.sigmoid(gi[:, H:2 * H] + gh[:, H:2 * H])
    n = jnp.tanh(gi[:, 2 * H:] + r * gh[:, 2 * H:])
    return (1.0 - z) * n + z * h


def _gru_c_body(x_ref, h_ref, wih_ref, whh_ref, bih_ref, bhh_ref, o_ref):
    x = x_ref[...]
    h = h_ref[...]
    gi = lax.dot_general(x, wih_ref[...], (((1,), (1,)), ((), ())),
                         preferred_element_type=jnp.float32) + bih_ref[...]
    gh = lax.dot_general(h, whh_ref[...], (((1,), (1,)), ((), ())),
                         preferred_element_type=jnp.float32) + bhh_ref[...]
    o_ref[...] = _gru_gates(gi, gh, h)


def _gru_c(x, h, Wih, Whh, bih, bhh):
    n = x.shape[0]
    return pl.pallas_call(
        _gru_c_body,
        grid=(n // BLK,),
        in_specs=[
            pl.BlockSpec((BLK, H), lambda i: (i, 0)),
            pl.BlockSpec((BLK, H), lambda i: (i, 0)),
            pl.BlockSpec((3 * H, H), lambda i: (0, 0)),
            pl.BlockSpec((3 * H, H), lambda i: (0, 0)),
            pl.BlockSpec((1, 3 * H), lambda i: (0, 0)),
            pl.BlockSpec((1, 3 * H), lambda i: (0, 0)),
        ],
        out_specs=pl.BlockSpec((BLK, H), lambda i: (i, 0)),
        out_shape=jax.ShapeDtypeStruct((n, H), jnp.float32),
        compiler_params=pltpu.CompilerParams(
            dimension_semantics=("parallel",)),
    )(x, h, Wih, Whh, bih, bhh)


def _gru_l_body(x1_ref, x2_ref, h_ref, wih_ref, whh_ref, bih_ref, bhh_ref,
                o_ref):
    h = h_ref[...]
    w = wih_ref[...]
    gi = (lax.dot_general(x1_ref[...], w[:, :H], (((1,), (1,)), ((), ())),
                          preferred_element_type=jnp.float32)
          + lax.dot_general(x2_ref[...], w[:, H:], (((1,), (1,)), ((), ())),
                            preferred_element_type=jnp.float32)
          + bih_ref[...])
    gh = lax.dot_general(h, whh_ref[...], (((1,), (1,)), ((), ())),
                         preferred_element_type=jnp.float32) + bhh_ref[...]
    o_ref[...] = _gru_gates(gi, gh, h)


def _gru_l(x1, l_emb, Wih, Whh, bih, bhh):
    n = l_emb.shape[0]
    nb = n // BLK
    return pl.pallas_call(
        _gru_l_body,
        grid=(nb,),
        in_specs=[
            pl.BlockSpec((BLK, H), lambda i: (i, 0)),
            pl.BlockSpec((BLK, H), lambda i: ((i + nb // 2) % nb, 0)),
            pl.BlockSpec((BLK, H), lambda i: (i, 0)),
            pl.BlockSpec((3 * H, 2 * H), lambda i: (0, 0)),
            pl.BlockSpec((3 * H, H), lambda i: (0, 0)),
            pl.BlockSpec((1, 3 * H), lambda i: (0, 0)),
            pl.BlockSpec((1, 3 * H), lambda i: (0, 0)),
        ],
        out_specs=pl.BlockSpec((BLK, H), lambda i: (i, 0)),
        out_shape=jax.ShapeDtypeStruct((n, H), jnp.float32),
        compiler_params=pltpu.CompilerParams(
            dimension_semantics=("parallel",)),
    )(x1, l_emb, l_emb, Wih, Whh, bih, bhh)


def kernel(l_embedding, c_embedding, pos_edge_index, neg_edge_index,
           l_mlp_W, l_mlp_b, c_mlp_W, c_mlp_b,
           l_gru_Wih, l_gru_Whh, l_gru_bih, l_gru_bhh,
           c_gru_Wih, c_gru_Whh, c_gru_bih, c_gru_bhh):
    ps = pos_edge_index[0].astype(jnp.int32)
    pd = pos_edge_index[1].astype(jnp.int32)
    ns = neg_edge_index[0].astype(jnp.int32)
    nd = neg_edge_index[1].astype(jnp.int32)

    def pad_pair(src, dst, n_pad):
        extra = n_pad - src.shape[0]
        src_p = jnp.concatenate([src, jnp.zeros((extra,), jnp.int32)])
        dst_p = jnp.concatenate([dst, jnp.full((extra,), SEG_OUT, jnp.int32)])
        return src_p, dst_p

    l2c_src, l2c_dst = pad_pair(
        jnp.concatenate([ps, ns + NUM_POS]), jnp.concatenate([pd, nd]), E2P)
    c2lp_src, c2lp_dst = pad_pair(pd, ps, E1P)
    c2ln_src, c2ln_dst = pad_pair(nd, ns, E1P)

    bih_l = l_gru_bih.reshape(1, 3 * H)
    bhh_l = l_gru_bhh.reshape(1, 3 * H)
    bih_c = c_gru_bih.reshape(1, 3 * H)
    bhh_c = c_gru_bhh.reshape(1, 3 * H)

    l_emb, c_emb = l_embedding, c_embedding
    for _ in range(NUM_ROUND):
        l_msg = _mlp(l_emb, l_mlp_W, l_mlp_b)
        c_msg = _mlp(c_emb, c_mlp_W, c_mlp_b)
        l2c = _seg_sum_2(l_msg, l2c_src, l2c_dst)
        c2l_p = _seg_sum_1(c_msg, c2lp_src, c2lp_dst)
        c2l_n = _seg_sum_1(c_msg, c2ln_src, c2ln_dst)
        c2l = jnp.concatenate([c2l_p, c2l_n], axis=0)
        c_emb = _gru_c(l2c, c_emb, c_gru_Wih, c_gru_Whh, bih_c, bhh_c)
        l_emb = _gru_l(c2l, l_emb, l_gru_Wih, l_gru_Whh, bih_l, bhh_l)
    return (l_emb, c_emb)
